# Initial kernel scaffold; baseline (speedup 1.0000x reference)
import jax
import jax.numpy as jnp
from jax.experimental import pallas as pl


def kernel(x, fmx):
    raise NotImplementedError("write your pallas kernel here")


# speedup vs baseline: 35.5965x; 35.5965x over previous
import functools

import jax
import jax.numpy as jnp
from jax import lax
from jax.experimental import pallas as pl
from jax.experimental.pallas import tpu as pltpu
from jax.experimental.pallas import tpu_sc as plsc

_C = 32
_S = 512
_N = 1048576
_B = 128
_L = 16


def _fv_body(gx_hbm, gy_hbm, tab_hbm, out_hbm,
             gxv_, gyv_, i00, i01, i10, i11, w00, w01, w10, w11,
             r00, r01, r10, r11, ov, sem):
    nc = 2
    wid = lax.axis_index("s") * nc + lax.axis_index("c")
    nw = 32
    spw = _N // nw
    nblk = spw // _B
    base = wid * spw

    span = float(_S - 1)
    half = 0.5 * span
    twos = 2.0 * span

    def compute_block(g, _):
        row0 = base + g * _B
        pltpu.sync_copy(gx_hbm.at[pl.ds(row0, _B)], gxv_)
        pltpu.sync_copy(gy_hbm.at[pl.ds(row0, _B)], gyv_)

        for i in range(_B // _L):
            l = i * _L
            gxv = gxv_[pl.ds(l, _L)]
            gyv = gyv_[pl.ds(l, _L)]

            def to_grid(gv):
                v = jnp.abs((gv + 1.0) * half)
                q = (v * (1.0 / twos)).astype(jnp.int32).astype(jnp.float32)
                v = v - q * twos
                v = jnp.where(v > span, twos - v, v)
                v = jnp.clip(v, 0.0, span)
                c0 = v.astype(jnp.int32)
                w = v - c0.astype(jnp.float32)
                c0 = jnp.clip(c0, 0, _S - 1)
                c1 = jnp.minimum(c0 + 1, _S - 1)
                return c0, c1, w

            x0, x1, wx = to_grid(gxv)
            y0, y1, wy = to_grid(gyv)

            yb0 = y0 * _S
            yb1 = y1 * _S
            sl = pl.ds(l, _L)
            i00[sl] = yb0 + x0
            i01[sl] = yb0 + x1
            i10[sl] = yb1 + x0
            i11[sl] = yb1 + x1
            wxc = 1.0 - wx
            wyc = 1.0 - wy
            w00[sl] = wxc * wyc
            w01[sl] = wx * wyc
            w10[sl] = wxc * wy
            w11[sl] = wx * wy

        c0 = pltpu.async_copy(tab_hbm.at[i00], r00, sem)
        c1 = pltpu.async_copy(tab_hbm.at[i01], r01, sem)
        c2 = pltpu.async_copy(tab_hbm.at[i10], r10, sem)
        c3 = pltpu.async_copy(tab_hbm.at[i11], r11, sem)
        c0.wait()
        c1.wait()
        c2.wait()
        c3.wait()

        lo = pl.ds(0, _L)
        hi = pl.ds(_L, _L)
        for i in range(_B // _L):
            l = i * _L
            wv00 = w00[pl.ds(l, _L)]
            wv01 = w01[pl.ds(l, _L)]
            wv10 = w10[pl.ds(l, _L)]
            wv11 = w11[pl.ds(l, _L)]
            for j in range(_L):
                b = l + j
                a00 = wv00[j]
                a01 = wv01[j]
                a10 = wv10[j]
                a11 = wv11[j]
                ov[b, lo] = (r00[b, lo] * a00 + r01[b, lo] * a01
                             + r10[b, lo] * a10 + r11[b, lo] * a11)
                ov[b, hi] = (r00[b, hi] * a00 + r01[b, hi] * a01
                             + r10[b, hi] * a10 + r11[b, hi] * a11)
        pltpu.sync_copy(ov, out_hbm.at[pl.ds(row0, _B)])
        return 0

    lax.fori_loop(0, nblk, compute_block, 0)


@jax.jit
def kernel(x, fmx):
    gx = x[:, 0]
    gy = x[:, 2]
    table = jnp.transpose(fmx[0], (1, 2, 0)).reshape(_S * _S, _C)
    mesh = plsc.VectorSubcoreMesh(core_axis_name="c", subcore_axis_name="s")
    fv = functools.partial(
        pl.kernel,
        mesh=mesh,
        compiler_params=pltpu.CompilerParams(use_tc_tiling_on_sc=False),
        out_type=jax.ShapeDtypeStruct((_N, _C), jnp.float32),
        scratch_types=[
            pltpu.VMEM((_B,), jnp.float32),
            pltpu.VMEM((_B,), jnp.float32),
            pltpu.VMEM((_B,), jnp.int32),
            pltpu.VMEM((_B,), jnp.int32),
            pltpu.VMEM((_B,), jnp.int32),
            pltpu.VMEM((_B,), jnp.int32),
            pltpu.VMEM((_B,), jnp.float32),
            pltpu.VMEM((_B,), jnp.float32),
            pltpu.VMEM((_B,), jnp.float32),
            pltpu.VMEM((_B,), jnp.float32),
            pltpu.VMEM((_B, _C), jnp.float32),
            pltpu.VMEM((_B, _C), jnp.float32),
            pltpu.VMEM((_B, _C), jnp.float32),
            pltpu.VMEM((_B, _C), jnp.float32),
            pltpu.VMEM((_B, _C), jnp.float32),
            pltpu.SemaphoreType.DMA,
        ],
    )(_fv_body)
    return fv(gx, gy, table)
